# baseline (speedup 1.0000x reference)
import functools

import jax
import jax.numpy as jnp
from jax import lax
from jax.experimental import pallas as pl
from jax.experimental.pallas import tpu as pltpu
from jax.experimental.pallas import tpu_sc as plsc

D = 128
H = 64
CHUNK = 64
NW = 32
NS = 16


def _absdiff_sc(x, src, dst):
    n_nodes = x.shape[0]
    es = src.shape[0]
    assert es % CHUNK == 0
    rows_per_tile = (n_nodes // NS) // 8 * 8
    tail = n_nodes - rows_per_tile * NS
    T = -(-es // (NW * CHUNK))
    T += T % 2
    mesh = plsc.VectorSubcoreMesh(core_axis_name="c", subcore_axis_name="s")

    @functools.partial(
        pl.kernel,
        mesh=mesh,
        out_type=jax.ShapeDtypeStruct((es, D), jnp.float32),
        scratch_types=[
            pltpu.VMEM_SHARED((n_nodes, D), jnp.float32),
            pltpu.VMEM((2, CHUNK), jnp.int32),
            pltpu.VMEM((2, CHUNK), jnp.int32),
            pltpu.VMEM((CHUNK, D), jnp.float32),
            pltpu.VMEM((CHUNK, D), jnp.float32),
            pltpu.VMEM((CHUNK, D), jnp.float32),
            pltpu.VMEM((CHUNK, D), jnp.float32),
            pltpu.SemaphoreType.DMA,
            pltpu.SemaphoreType.DMA,
            pltpu.SemaphoreType.DMA,
            pltpu.SemaphoreType.DMA,
            pltpu.SemaphoreType.DMA,
            pltpu.SemaphoreType.DMA,
        ],
    )
    def sc_kernel(x_hbm, src_hbm, dst_hbm, out_hbm,
                  x_sh, idx_s, idx_d, rs0, rs1, rd0, rd1,
                  si0, si1, sg0, sg1, sw0, sw1):
        sid = lax.axis_index("s")
        wid = sid * 2 + lax.axis_index("c")
        rows_s = (rs0, rs1)
        rows_d = (rd0, rd1)
        sem_i = (si0, si1)
        sem_g = (sg0, sg1)
        sem_w = (sw0, sw1)

        stage = pl.multiple_of(sid * rows_per_tile, 8)
        pltpu.sync_copy(x_hbm.at[pl.ds(stage, rows_per_tile)],
                        x_sh.at[pl.ds(stage, rows_per_tile)])
        if tail:
            @pl.when(sid == 0)
            def _():
                base = rows_per_tile * NS
                pltpu.sync_copy(x_hbm.at[pl.ds(base, tail)],
                                x_sh.at[pl.ds(base, tail)])
        plsc.subcore_barrier()

        def base_of(t):
            return pl.multiple_of(
                jnp.minimum((wid * T + t) * CHUNK, es - CHUNK), CHUNK)

        def start_idx(t, b):
            base = base_of(t)
            pltpu.async_copy(src_hbm.at[pl.ds(base, CHUNK)], idx_s.at[b],
                             sem_i[b])
            pltpu.async_copy(dst_hbm.at[pl.ds(base, CHUNK)], idx_d.at[b],
                             sem_i[b])

        def wait_idx(b):
            pltpu.make_async_copy(src_hbm.at[pl.ds(0, CHUNK)], idx_s.at[b],
                                  sem_i[b]).wait()
            pltpu.make_async_copy(dst_hbm.at[pl.ds(0, CHUNK)], idx_d.at[b],
                                  sem_i[b]).wait()

        def start_gather(b):
            pltpu.async_copy(x_sh.at[idx_s.at[b]], rows_s[b], sem_g[b])
            pltpu.async_copy(x_sh.at[idx_d.at[b]], rows_d[b], sem_g[b])

        def wait_gather(b):
            pltpu.make_async_copy(x_sh.at[idx_s.at[b]], rows_s[b],
                                  sem_g[b]).wait()
            pltpu.make_async_copy(x_sh.at[idx_d.at[b]], rows_d[b],
                                  sem_g[b]).wait()

        def start_write(t, b):
            pltpu.async_copy(rows_s[b], out_hbm.at[pl.ds(base_of(t), CHUNK)],
                             sem_w[b])

        def wait_write(b):
            pltpu.make_async_copy(rows_s[b], out_hbm.at[pl.ds(0, CHUNK)],
                                  sem_w[b]).wait()

        start_idx(0, 0)
        start_idx(1, 1)
        wait_idx(0)
        start_gather(0)

        def half_body(t, b):
            nb = 1 - b
            wait_idx(nb)

            @pl.when(t > 0)
            def _():
                wait_write(nb)

            start_gather(nb)
            wait_gather(b)
            start_idx(t + 2, b)

            def row_body(i, c):
                for k in range(D // 16):
                    sl = (i, pl.ds(k * 16, 16))
                    rows_s[b][sl] = jnp.abs(rows_s[b][sl] - rows_d[b][sl])
                return c

            lax.fori_loop(0, CHUNK, row_body, 0, unroll=4)
            start_write(t, b)

        def pair_body(i, carry):
            half_body(2 * i, 0)
            half_body(2 * i + 1, 1)
            return carry

        lax.fori_loop(0, T // 2, pair_body, 0)

        wait_idx(1)
        wait_gather(0)
        wait_write(1)

    return sc_kernel(x, src, dst)


def _mlp_tc(dif, W1, b1, W2, b2, acc, off, e, be):
    es = dif.shape[0]
    assert es % be == 0 and off % be == 0

    def body(d_ref, w1_ref, b1_ref, w2_ref, b2_ref, *rest):
        o_ref = rest[-1]
        h = jnp.dot(d_ref[...], w1_ref[...], preferred_element_type=jnp.float32)
        h = jnp.maximum(h + b1_ref[...], 0.0)
        o_ref[...] = (
            jnp.dot(h, w2_ref[...], preferred_element_type=jnp.float32)
            + b2_ref[...]
        )

    ob = off // be
    in_specs = [
        pl.BlockSpec((be, D), lambda i: (i, 0)),
        pl.BlockSpec((D, H), lambda i: (0, 0)),
        pl.BlockSpec((1, H), lambda i: (0, 0)),
        pl.BlockSpec((H, 2), lambda i: (0, 0)),
        pl.BlockSpec((1, 2), lambda i: (0, 0)),
    ]
    args = [dif, W1, b1.reshape(1, H), W2, b2.reshape(1, 2)]
    aliases = {}
    if acc is not None:
        in_specs.append(pl.BlockSpec(memory_space=pl.ANY))
        args.append(acc)
        aliases = {5: 0}
    return pl.pallas_call(
        body,
        grid=(es // be,),
        in_specs=in_specs,
        out_specs=pl.BlockSpec((be, 2), lambda i: (i + ob, 0)),
        out_shape=jax.ShapeDtypeStruct((e, 2), jnp.float32),
        input_output_aliases=aliases,
    )(*args)


def kernel(x, edge_index, W1, b1, W2, b2):
    e = edge_index.shape[1]
    slabs = [80000, 80000, 80000, 80000] if e == 320000 else [e]
    assert sum(slabs) == e
    acc = None
    off = 0
    for es in slabs:
        src = lax.slice_in_dim(edge_index[0], off, off + es)
        dst = lax.slice_in_dim(edge_index[1], off, off + es)
        dif = _absdiff_sc(x, src, dst)
        acc = _mlp_tc(dif, W1, b1, W2, b2, acc, off, e, be=8000)
        off += es
    return acc


# speedup vs baseline: 1.0481x; 1.0481x over previous
import functools

import jax
import jax.numpy as jnp
from jax import lax
from jax.experimental import pallas as pl
from jax.experimental.pallas import tpu as pltpu
from jax.experimental.pallas import tpu_sc as plsc

D = 128
H = 64
CHUNK = 64
NW = 32
NS = 16


def _absdiff_sc(x, src, dst):
    n_nodes = x.shape[0]
    es = src.shape[0]
    assert es % CHUNK == 0
    rows_per_tile = (n_nodes // NS) // 8 * 8
    tail = n_nodes - rows_per_tile * NS
    T = -(-es // (NW * CHUNK))
    T += T % 2
    mesh = plsc.VectorSubcoreMesh(core_axis_name="c", subcore_axis_name="s")

    @functools.partial(
        pl.kernel,
        mesh=mesh,
        out_type=jax.ShapeDtypeStruct((es, D), jnp.float32),
        scratch_types=[
            pltpu.VMEM_SHARED((n_nodes, D), jnp.float32),
            pltpu.VMEM((2, CHUNK), jnp.int32),
            pltpu.VMEM((2, CHUNK), jnp.int32),
            pltpu.VMEM((CHUNK, D), jnp.float32),
            pltpu.VMEM((CHUNK, D), jnp.float32),
            pltpu.VMEM((CHUNK, D), jnp.float32),
            pltpu.VMEM((CHUNK, D), jnp.float32),
            pltpu.SemaphoreType.DMA,
            pltpu.SemaphoreType.DMA,
            pltpu.SemaphoreType.DMA,
            pltpu.SemaphoreType.DMA,
            pltpu.SemaphoreType.DMA,
            pltpu.SemaphoreType.DMA,
        ],
    )
    def sc_kernel(x_hbm, src_hbm, dst_hbm, out_hbm,
                  x_sh, idx_s, idx_d, rs0, rs1, rd0, rd1,
                  si0, si1, sg0, sg1, sw0, sw1):
        sid = lax.axis_index("s")
        wid = sid * 2 + lax.axis_index("c")
        rows_s = (rs0, rs1)
        rows_d = (rd0, rd1)
        sem_i = (si0, si1)
        sem_g = (sg0, sg1)
        sem_w = (sw0, sw1)

        stage = pl.multiple_of(sid * rows_per_tile, 8)
        pltpu.sync_copy(x_hbm.at[pl.ds(stage, rows_per_tile)],
                        x_sh.at[pl.ds(stage, rows_per_tile)])
        if tail:
            @pl.when(sid == 0)
            def _():
                base = rows_per_tile * NS
                pltpu.sync_copy(x_hbm.at[pl.ds(base, tail)],
                                x_sh.at[pl.ds(base, tail)])
        plsc.subcore_barrier()

        def base_of(t):
            return pl.multiple_of(
                jnp.minimum((wid * T + t) * CHUNK, es - CHUNK), CHUNK)

        def start_idx(t, b):
            base = base_of(t)
            pltpu.async_copy(src_hbm.at[pl.ds(base, CHUNK)], idx_s.at[b],
                             sem_i[b])
            pltpu.async_copy(dst_hbm.at[pl.ds(base, CHUNK)], idx_d.at[b],
                             sem_i[b])

        def wait_idx(b):
            pltpu.make_async_copy(src_hbm.at[pl.ds(0, CHUNK)], idx_s.at[b],
                                  sem_i[b]).wait()
            pltpu.make_async_copy(dst_hbm.at[pl.ds(0, CHUNK)], idx_d.at[b],
                                  sem_i[b]).wait()

        def start_gather(b):
            pltpu.async_copy(x_sh.at[idx_s.at[b]], rows_s[b], sem_g[b])
            pltpu.async_copy(x_sh.at[idx_d.at[b]], rows_d[b], sem_g[b])

        def wait_gather(b):
            pltpu.make_async_copy(x_sh.at[idx_s.at[b]], rows_s[b],
                                  sem_g[b]).wait()
            pltpu.make_async_copy(x_sh.at[idx_d.at[b]], rows_d[b],
                                  sem_g[b]).wait()

        def start_write(t, b):
            pltpu.async_copy(rows_s[b], out_hbm.at[pl.ds(base_of(t), CHUNK)],
                             sem_w[b])

        def wait_write(b):
            pltpu.make_async_copy(rows_s[b], out_hbm.at[pl.ds(0, CHUNK)],
                                  sem_w[b]).wait()

        start_idx(0, 0)
        start_idx(1, 1)
        wait_idx(0)
        start_gather(0)

        def half_body(t, b):
            nb = 1 - b
            wait_idx(nb)

            @pl.when(t > 0)
            def _():
                wait_write(nb)

            start_gather(nb)
            wait_gather(b)
            start_idx(t + 2, b)

            def row_body(i, c):
                for k in range(D // 16):
                    sl = (i, pl.ds(k * 16, 16))
                    rows_s[b][sl] = jnp.abs(rows_s[b][sl] - rows_d[b][sl])
                return c

            lax.fori_loop(0, CHUNK, row_body, 0, unroll=4)
            start_write(t, b)

        def pair_body(i, carry):
            half_body(2 * i, 0)
            half_body(2 * i + 1, 1)
            return carry

        lax.fori_loop(0, T // 2, pair_body, 0)

        wait_idx(1)
        wait_gather(0)
        wait_write(1)

    return sc_kernel(x, src, dst)


def _mlp_tc(dif, W1, b1, W2, b2, acc, off, e, be):
    es = dif.shape[0]
    assert es % be == 0 and off % be == 0

    def body(d_ref, w1_ref, b1_ref, w2_ref, b2_ref, *rest):
        o_ref = rest[-1]
        h = jnp.dot(d_ref[...], w1_ref[...], preferred_element_type=jnp.float32)
        h = jnp.maximum(h + b1_ref[...], 0.0)
        o_ref[...] = (
            jnp.dot(h, w2_ref[...], preferred_element_type=jnp.float32)
            + b2_ref[...]
        )

    ob = off // be
    in_specs = [
        pl.BlockSpec((be, D), lambda i: (i, 0)),
        pl.BlockSpec((D, H), lambda i: (0, 0)),
        pl.BlockSpec((1, H), lambda i: (0, 0)),
        pl.BlockSpec((H, 2), lambda i: (0, 0)),
        pl.BlockSpec((1, 2), lambda i: (0, 0)),
    ]
    args = [dif, W1, b1.reshape(1, H), W2, b2.reshape(1, 2)]
    aliases = {}
    if acc is not None:
        in_specs.append(pl.BlockSpec(memory_space=pl.ANY))
        args.append(acc)
        aliases = {5: 0}
    return pl.pallas_call(
        body,
        grid=(es // be,),
        in_specs=in_specs,
        out_specs=pl.BlockSpec((be, 2), lambda i: (i + ob, 0)),
        out_shape=jax.ShapeDtypeStruct((e, 2), jnp.float32),
        input_output_aliases=aliases,
    )(*args)


def kernel(x, edge_index, W1, b1, W2, b2):
    e = edge_index.shape[1]
    slabs = [80000, 80000, 80000, 80000] if e == 320000 else [e]
    assert sum(slabs) == e
    outs = []
    off = 0
    for es in slabs:
        src = lax.slice_in_dim(edge_index[0], off, off + es)
        dst = lax.slice_in_dim(edge_index[1], off, off + es)
        dif = _absdiff_sc(x, src, dst)
        outs.append(_mlp_tc(dif, W1, b1, W2, b2, None, 0, es, be=8000))
        off += es
    return outs[0] if len(outs) == 1 else jnp.concatenate(outs, axis=0)


# speedup vs baseline: 1.0491x; 1.0009x over previous
import functools

import jax
import jax.numpy as jnp
from jax import lax
from jax.experimental import pallas as pl
from jax.experimental.pallas import tpu as pltpu
from jax.experimental.pallas import tpu_sc as plsc

D = 128
H = 64
CHUNK = 80
NW = 32
NS = 16


def _absdiff_sc(x, src, dst):
    n_nodes = x.shape[0]
    es = src.shape[0]
    assert es % CHUNK == 0
    rows_per_tile = (n_nodes // NS) // 8 * 8
    tail = n_nodes - rows_per_tile * NS
    T = -(-es // (NW * CHUNK))
    T += T % 2
    mesh = plsc.VectorSubcoreMesh(core_axis_name="c", subcore_axis_name="s")

    @functools.partial(
        pl.kernel,
        mesh=mesh,
        out_type=jax.ShapeDtypeStruct((es, D), jnp.float32),
        scratch_types=[
            pltpu.VMEM_SHARED((n_nodes, D), jnp.float32),
            pltpu.VMEM((2, CHUNK), jnp.int32),
            pltpu.VMEM((2, CHUNK), jnp.int32),
            pltpu.VMEM((CHUNK, D), jnp.float32),
            pltpu.VMEM((CHUNK, D), jnp.float32),
            pltpu.VMEM((CHUNK, D), jnp.float32),
            pltpu.VMEM((CHUNK, D), jnp.float32),
            pltpu.SemaphoreType.DMA,
            pltpu.SemaphoreType.DMA,
            pltpu.SemaphoreType.DMA,
            pltpu.SemaphoreType.DMA,
            pltpu.SemaphoreType.DMA,
            pltpu.SemaphoreType.DMA,
        ],
    )
    def sc_kernel(x_hbm, src_hbm, dst_hbm, out_hbm,
                  x_sh, idx_s, idx_d, rs0, rs1, rd0, rd1,
                  si0, si1, sg0, sg1, sw0, sw1):
        sid = lax.axis_index("s")
        wid = sid * 2 + lax.axis_index("c")
        rows_s = (rs0, rs1)
        rows_d = (rd0, rd1)
        sem_i = (si0, si1)
        sem_g = (sg0, sg1)
        sem_w = (sw0, sw1)

        stage = pl.multiple_of(sid * rows_per_tile, 8)
        pltpu.sync_copy(x_hbm.at[pl.ds(stage, rows_per_tile)],
                        x_sh.at[pl.ds(stage, rows_per_tile)])
        if tail:
            @pl.when(sid == 0)
            def _():
                base = rows_per_tile * NS
                pltpu.sync_copy(x_hbm.at[pl.ds(base, tail)],
                                x_sh.at[pl.ds(base, tail)])
        plsc.subcore_barrier()

        def base_of(t):
            return pl.multiple_of(
                jnp.minimum((wid * T + t) * CHUNK, es - CHUNK), CHUNK)

        def start_idx(t, b):
            base = base_of(t)
            pltpu.async_copy(src_hbm.at[pl.ds(base, CHUNK)], idx_s.at[b],
                             sem_i[b])
            pltpu.async_copy(dst_hbm.at[pl.ds(base, CHUNK)], idx_d.at[b],
                             sem_i[b])

        def wait_idx(b):
            pltpu.make_async_copy(src_hbm.at[pl.ds(0, CHUNK)], idx_s.at[b],
                                  sem_i[b]).wait()
            pltpu.make_async_copy(dst_hbm.at[pl.ds(0, CHUNK)], idx_d.at[b],
                                  sem_i[b]).wait()

        def start_gather(b):
            pltpu.async_copy(x_sh.at[idx_s.at[b]], rows_s[b], sem_g[b])
            pltpu.async_copy(x_sh.at[idx_d.at[b]], rows_d[b], sem_g[b])

        def wait_gather(b):
            pltpu.make_async_copy(x_sh.at[idx_s.at[b]], rows_s[b],
                                  sem_g[b]).wait()
            pltpu.make_async_copy(x_sh.at[idx_d.at[b]], rows_d[b],
                                  sem_g[b]).wait()

        def start_write(t, b):
            pltpu.async_copy(rows_s[b], out_hbm.at[pl.ds(base_of(t), CHUNK)],
                             sem_w[b])

        def wait_write(b):
            pltpu.make_async_copy(rows_s[b], out_hbm.at[pl.ds(0, CHUNK)],
                                  sem_w[b]).wait()

        start_idx(0, 0)
        start_idx(1, 1)
        wait_idx(0)
        start_gather(0)

        def half_body(t, b):
            nb = 1 - b
            wait_idx(nb)

            @pl.when(t > 0)
            def _():
                wait_write(nb)

            start_gather(nb)
            wait_gather(b)
            start_idx(t + 2, b)

            def row_body(i, c):
                for k in range(D // 16):
                    sl = (i, pl.ds(k * 16, 16))
                    rows_s[b][sl] = jnp.abs(rows_s[b][sl] - rows_d[b][sl])
                return c

            lax.fori_loop(0, CHUNK, row_body, 0, unroll=4)
            start_write(t, b)

        def pair_body(i, carry):
            half_body(2 * i, 0)
            half_body(2 * i + 1, 1)
            return carry

        lax.fori_loop(0, T // 2, pair_body, 0)

        wait_idx(1)
        wait_gather(0)
        wait_write(1)

    return sc_kernel(x, src, dst)


def _mlp_tc(dif, W1, b1, W2, b2, acc, off, e, be):
    es = dif.shape[0]
    assert es % be == 0 and off % be == 0

    def body(d_ref, w1_ref, b1_ref, w2_ref, b2_ref, *rest):
        o_ref = rest[-1]
        h = jnp.dot(d_ref[...], w1_ref[...], preferred_element_type=jnp.float32)
        h = jnp.maximum(h + b1_ref[...], 0.0)
        o_ref[...] = (
            jnp.dot(h, w2_ref[...], preferred_element_type=jnp.float32)
            + b2_ref[...]
        )

    ob = off // be
    in_specs = [
        pl.BlockSpec((be, D), lambda i: (i, 0)),
        pl.BlockSpec((D, H), lambda i: (0, 0)),
        pl.BlockSpec((1, H), lambda i: (0, 0)),
        pl.BlockSpec((H, 2), lambda i: (0, 0)),
        pl.BlockSpec((1, 2), lambda i: (0, 0)),
    ]
    args = [dif, W1, b1.reshape(1, H), W2, b2.reshape(1, 2)]
    aliases = {}
    if acc is not None:
        in_specs.append(pl.BlockSpec(memory_space=pl.ANY))
        args.append(acc)
        aliases = {5: 0}
    return pl.pallas_call(
        body,
        grid=(es // be,),
        in_specs=in_specs,
        out_specs=pl.BlockSpec((be, 2), lambda i: (i + ob, 0)),
        out_shape=jax.ShapeDtypeStruct((e, 2), jnp.float32),
        input_output_aliases=aliases,
    )(*args)


def kernel(x, edge_index, W1, b1, W2, b2):
    e = edge_index.shape[1]
    slabs = [80000, 80000, 80000, 80000] if e == 320000 else [e]
    assert sum(slabs) == e
    outs = []
    off = 0
    for es in slabs:
        src = lax.slice_in_dim(edge_index[0], off, off + es)
        dst = lax.slice_in_dim(edge_index[1], off, off + es)
        dif = _absdiff_sc(x, src, dst)
        outs.append(_mlp_tc(dif, W1, b1, W2, b2, None, 0, es, be=8000))
        off += es
    return outs[0] if len(outs) == 1 else jnp.concatenate(outs, axis=0)


# speedup vs baseline: 1.9848x; 1.8919x over previous
import functools

import jax
import jax.numpy as jnp
from jax import lax
from jax.experimental import pallas as pl
from jax.experimental.pallas import tpu as pltpu
from jax.experimental.pallas import tpu_sc as plsc

D = 128
H = 64
CHUNK = 64
NW = 32
NS = 16


def _absdiff_sc(x, src, dst):
    n_nodes = x.shape[0]
    es = src.shape[0]
    assert es % CHUNK == 0
    rows_per_tile = (n_nodes // NS) // 8 * 8
    tail = n_nodes - rows_per_tile * NS
    T = -(-es // (NW * CHUNK))
    T += T % 2
    mesh = plsc.VectorSubcoreMesh(core_axis_name="c", subcore_axis_name="s")

    @functools.partial(
        pl.kernel,
        mesh=mesh,
        out_type=jax.ShapeDtypeStruct((es, D), jnp.float32),
        scratch_types=[
            pltpu.VMEM_SHARED((n_nodes, D), jnp.float32),
            pltpu.VMEM((2, 2 * CHUNK), jnp.int32),
            pltpu.VMEM((2 * CHUNK, D), jnp.float32),
            pltpu.VMEM((2 * CHUNK, D), jnp.float32),
            pltpu.SemaphoreType.DMA,
            pltpu.SemaphoreType.DMA,
            pltpu.SemaphoreType.DMA,
            pltpu.SemaphoreType.DMA,
            pltpu.SemaphoreType.DMA,
            pltpu.SemaphoreType.DMA,
        ],
    )
    def sc_kernel(x_hbm, src_hbm, dst_hbm, out_hbm,
                  x_sh, idx, r0, r1,
                  si0, si1, sg0, sg1, sw0, sw1):
        sid = lax.axis_index("s")
        wid = sid * 2 + lax.axis_index("c")
        rows = (r0, r1)
        sem_i = (si0, si1)
        sem_g = (sg0, sg1)
        sem_w = (sw0, sw1)

        stage = pl.multiple_of(sid * rows_per_tile, 8)
        pltpu.sync_copy(x_hbm.at[pl.ds(stage, rows_per_tile)],
                        x_sh.at[pl.ds(stage, rows_per_tile)])
        if tail:
            @pl.when(sid == 0)
            def _():
                base = rows_per_tile * NS
                pltpu.sync_copy(x_hbm.at[pl.ds(base, tail)],
                                x_sh.at[pl.ds(base, tail)])
        plsc.subcore_barrier()

        def base_of(t):
            return pl.multiple_of(
                jnp.minimum((wid * T + t) * CHUNK, es - CHUNK), CHUNK)

        def start_idx(t, b):
            base = base_of(t)
            pltpu.async_copy(src_hbm.at[pl.ds(base, CHUNK)],
                             idx.at[b, pl.ds(0, CHUNK)], sem_i[b])
            pltpu.async_copy(dst_hbm.at[pl.ds(base, CHUNK)],
                             idx.at[b, pl.ds(CHUNK, CHUNK)], sem_i[b])

        def wait_idx(b):
            pltpu.make_async_copy(src_hbm.at[pl.ds(0, CHUNK)],
                                  idx.at[b, pl.ds(0, CHUNK)], sem_i[b]).wait()
            pltpu.make_async_copy(src_hbm.at[pl.ds(0, CHUNK)],
                                  idx.at[b, pl.ds(0, CHUNK)], sem_i[b]).wait()

        def start_gather(b):
            pltpu.async_copy(x_sh.at[idx.at[b]], rows[b], sem_g[b])

        def wait_gather(b):
            pltpu.make_async_copy(x_sh.at[idx.at[b]], rows[b],
                                  sem_g[b]).wait()

        def start_write(t, b):
            pltpu.async_copy(rows[b].at[pl.ds(0, CHUNK)],
                             out_hbm.at[pl.ds(base_of(t), CHUNK)], sem_w[b])

        def wait_write(b):
            pltpu.make_async_copy(rows[b].at[pl.ds(0, CHUNK)],
                                  out_hbm.at[pl.ds(0, CHUNK)], sem_w[b]).wait()

        start_idx(0, 0)
        start_idx(1, 1)
        wait_idx(0)
        start_gather(0)

        def half_body(t, b):
            nb = 1 - b
            wait_idx(nb)

            @pl.when(t > 0)
            def _():
                wait_write(nb)

            start_gather(nb)
            wait_gather(b)
            start_idx(t + 2, b)

            def row_body(i, c):
                for k in range(D // 16):
                    sl = (i, pl.ds(k * 16, 16))
                    sd = (CHUNK + i, pl.ds(k * 16, 16))
                    rows[b][sl] = jnp.abs(rows[b][sl] - rows[b][sd])
                return c

            lax.fori_loop(0, CHUNK, row_body, 0, unroll=4)
            start_write(t, b)

        def pair_body(i, carry):
            half_body(2 * i, 0)
            half_body(2 * i + 1, 1)
            return carry

        lax.fori_loop(0, T // 2, pair_body, 0)

        wait_idx(1)
        wait_gather(0)
        wait_write(1)

    return sc_kernel(x, src, dst)


def _mlp_tc(dif, W1, b1, W2, b2, acc, off, e, be):
    es = dif.shape[0]
    assert es % be == 0 and off % be == 0

    def body(d_ref, w1_ref, b1_ref, w2_ref, b2_ref, *rest):
        o_ref = rest[-1]
        h = jnp.dot(d_ref[...], w1_ref[...], preferred_element_type=jnp.float32)
        h = jnp.maximum(h + b1_ref[...], 0.0)
        o_ref[...] = (
            jnp.dot(h, w2_ref[...], preferred_element_type=jnp.float32)
            + b2_ref[...]
        )

    ob = off // be
    in_specs = [
        pl.BlockSpec((be, D), lambda i: (i, 0)),
        pl.BlockSpec((D, H), lambda i: (0, 0)),
        pl.BlockSpec((1, H), lambda i: (0, 0)),
        pl.BlockSpec((H, 2), lambda i: (0, 0)),
        pl.BlockSpec((1, 2), lambda i: (0, 0)),
    ]
    args = [dif, W1, b1.reshape(1, H), W2, b2.reshape(1, 2)]
    aliases = {}
    if acc is not None:
        in_specs.append(pl.BlockSpec(memory_space=pl.ANY))
        args.append(acc)
        aliases = {5: 0}
    return pl.pallas_call(
        body,
        grid=(es // be,),
        in_specs=in_specs,
        out_specs=pl.BlockSpec((be, 2), lambda i: (i + ob, 0)),
        out_shape=jax.ShapeDtypeStruct((e, 2), jnp.float32),
        input_output_aliases=aliases,
    )(*args)


def kernel(x, edge_index, W1, b1, W2, b2):
    e = edge_index.shape[1]
    slabs = [80000, 80000, 80000, 80000] if e == 320000 else [e]
    assert sum(slabs) == e
    outs = []
    off = 0
    for es in slabs:
        src = lax.slice_in_dim(edge_index[0], off, off + es)
        dst = lax.slice_in_dim(edge_index[1], off, off + es)
        dif = _absdiff_sc(x, src, dst)
        outs.append(_mlp_tc(dif, W1, b1, W2, b2, None, 0, es, be=8000))
        off += es
    return outs[0] if len(outs) == 1 else jnp.concatenate(outs, axis=0)
